# baseline (speedup 1.0000x reference)
import jax
import jax.numpy as jnp
from jax.experimental import pallas as pl
from jax.experimental.pallas import tpu as pltpu

BM = 400


def _support_body(x_ref, w_ref, s_ref):
    s_ref[...] = jnp.dot(
        x_ref[...], w_ref[...], preferred_element_type=jnp.float32
    )


def _main_body(s_ref, b_ref, adj_ref, out_ref):
    acc = jnp.dot(
        adj_ref[...], s_ref[...], preferred_element_type=jnp.float32
    )
    out_ref[...] = acc + b_ref[...]


def kernel(input, adj, weight, bias):
    n, f_in = input.shape
    f_out = weight.shape[1]
    support = pl.pallas_call(
        _support_body,
        out_shape=jax.ShapeDtypeStruct((n, f_out), jnp.float32),
    )(input, weight)
    return pl.pallas_call(
        _main_body,
        grid=(n // BM,),
        in_specs=[
            pl.BlockSpec((n, f_out), lambda i: (0, 0)),
            pl.BlockSpec((1, f_out), lambda i: (0, 0)),
            pl.BlockSpec((BM, n), lambda i: (i, 0)),
        ],
        out_specs=pl.BlockSpec((BM, f_out), lambda i: (i, 0)),
        out_shape=jax.ShapeDtypeStruct((n, f_out), jnp.float32),
        compiler_params=pltpu.CompilerParams(
            dimension_semantics=("parallel",),
            vmem_limit_bytes=64 * 1024 * 1024,
        ),
    )(support, bias.reshape(1, f_out), adj)


# speedup vs baseline: 1.0174x; 1.0174x over previous
import jax
import jax.numpy as jnp
from jax.experimental import pallas as pl
from jax.experimental.pallas import tpu as pltpu

BM = 200
NBUF = 4


def _body(x_ref, w_ref, b_ref, adj_ref, out_ref,
          support_ref, bufs, obufs, in_sems, out_sems):
    n = adj_ref.shape[0]
    nb = n // BM

    support_ref[...] = jnp.dot(
        x_ref[...], w_ref[...], preferred_element_type=jnp.float32
    )

    def in_copy(blk, slot):
        return pltpu.make_async_copy(
            adj_ref.at[pl.ds(blk * BM, BM), :], bufs.at[slot],
            in_sems.at[slot],
        )

    def out_copy(blk, oslot):
        return pltpu.make_async_copy(
            obufs.at[oslot], out_ref.at[pl.ds(blk * BM, BM), :],
            out_sems.at[oslot],
        )

    for s in range(NBUF):
        in_copy(s, s).start()

    def step(i, carry):
        slot = jax.lax.rem(i, NBUF)
        oslot = jax.lax.rem(i, 2)
        in_copy(i, slot).wait()
        acc = jnp.dot(
            bufs[slot], support_ref[...], preferred_element_type=jnp.float32
        )

        @pl.when(i >= 2)
        def _():
            out_copy(i - 2, oslot).wait()

        obufs[oslot] = acc + b_ref[...]
        out_copy(i, oslot).start()

        @pl.when(i + NBUF < nb)
        def _():
            in_copy(i + NBUF, slot).start()

        return carry

    jax.lax.fori_loop(0, nb, step, 0)
    out_copy(nb - 2, (nb - 2) % 2).wait()
    out_copy(nb - 1, (nb - 1) % 2).wait()


def kernel(input, adj, weight, bias):
    n, f_in = input.shape
    f_out = weight.shape[1]
    return pl.pallas_call(
        _body,
        in_specs=[
            pl.BlockSpec((n, f_in), lambda: (0, 0)),
            pl.BlockSpec((f_in, f_out), lambda: (0, 0)),
            pl.BlockSpec((1, f_out), lambda: (0, 0)),
            pl.BlockSpec(memory_space=pl.ANY),
        ],
        out_specs=pl.BlockSpec(memory_space=pl.ANY),
        out_shape=jax.ShapeDtypeStruct((n, f_out), jnp.float32),
        scratch_shapes=[
            pltpu.VMEM((n, f_out), jnp.float32),
            pltpu.VMEM((NBUF, BM, n), jnp.float32),
            pltpu.VMEM((2, BM, f_out), jnp.float32),
            pltpu.SemaphoreType.DMA((NBUF,)),
            pltpu.SemaphoreType.DMA((2,)),
        ],
        compiler_params=pltpu.CompilerParams(
            vmem_limit_bytes=64 * 1024 * 1024,
        ),
    )(input, weight, bias.reshape(1, f_out), adj)


# speedup vs baseline: 1.0439x; 1.0261x over previous
import jax
import jax.numpy as jnp
from jax.experimental import pallas as pl
from jax.experimental.pallas import tpu as pltpu

BM = 400


def _body(x_ref, w_ref, b_ref, adj_ref, out_ref, support_ref):
    @pl.when(pl.program_id(0) == 0)
    def _():
        support_ref[...] = jnp.dot(
            x_ref[...], w_ref[...], preferred_element_type=jnp.float32
        )

    acc = jnp.dot(
        adj_ref[...], support_ref[...], preferred_element_type=jnp.float32
    )
    out_ref[...] = acc + b_ref[...]


def kernel(input, adj, weight, bias):
    n, f_in = input.shape
    f_out = weight.shape[1]
    grid = (n // BM,)
    return pl.pallas_call(
        _body,
        grid=grid,
        in_specs=[
            pl.BlockSpec((n, f_in), lambda i: (0, 0)),
            pl.BlockSpec((f_in, f_out), lambda i: (0, 0)),
            pl.BlockSpec((1, f_out), lambda i: (0, 0)),
            pl.BlockSpec((BM, n), lambda i: (i, 0)),
        ],
        out_specs=pl.BlockSpec((BM, f_out), lambda i: (i, 0)),
        out_shape=jax.ShapeDtypeStruct((n, f_out), jnp.float32),
        scratch_shapes=[pltpu.VMEM((n, f_out), jnp.float32)],
        compiler_params=pltpu.CompilerParams(
            dimension_semantics=("arbitrary",),
            vmem_limit_bytes=64 * 1024 * 1024,
        ),
    )(input, weight, bias.reshape(1, f_out), adj)
